# baseline (speedup 1.0000x reference)
import jax
import jax.numpy as jnp
from jax.experimental import pallas as pl
from jax.experimental.pallas import tpu as pltpu

_BI = 512


def _sage_block(adjl_ref, adjr_ref, h_ref, wt_ref, out_ref):
    nh = h_ref.shape[0] // 2
    al = jnp.minimum(adjl_ref[...], 1.0)
    ar = jnp.minimum(adjr_ref[...], 1.0)
    x = jnp.dot(al, h_ref[:nh, :], preferred_element_type=jnp.float32)
    x = x + jnp.dot(ar, h_ref[nh:, :], preferred_element_type=jnp.float32)
    out_ref[...] = jnp.dot(x, wt_ref[...], preferred_element_type=jnp.float32)


def kernel(h, adj, W):
    n, d_in = h.shape
    d_out = W.shape[0]
    wt = W.T
    nh = n // 2
    grid = (n // _BI,)
    return pl.pallas_call(
        _sage_block,
        grid=grid,
        in_specs=[
            pl.BlockSpec((_BI, nh), lambda i: (i, 0)),
            pl.BlockSpec((_BI, nh), lambda i: (i, 1)),
            pl.BlockSpec((n, d_in), lambda i: (0, 0)),
            pl.BlockSpec((d_in, d_out), lambda i: (0, 0)),
        ],
        out_specs=pl.BlockSpec((_BI, d_out), lambda i: (i, 0)),
        out_shape=jax.ShapeDtypeStruct((n, d_out), jnp.float32),
        compiler_params=pltpu.CompilerParams(
            dimension_semantics=("arbitrary",),
        ),
    )(adj, adj, h, wt)
